# baseline (speedup 1.0000x reference)
import jax
import jax.numpy as jnp
from jax import lax
from jax.experimental import pallas as pl
from jax.experimental.pallas import tpu as pltpu
from jax.experimental.pallas import tpu_sc as plsc

_NUM_TYPES = 16
_HIDDEN = 512
_EMBED = 768
_B = 16384

_NC = 2
_NS = 16
_NW = _NC * _NS
_BPW = _B // _NW
_NBUF = 2
_CHUNK = 64
_NCHUNK = _BPW // _CHUNK
_LANES = 16
_NSEG = _EMBED // _LANES


def _mlp_body(table_ref, w1_ref, b1_ref, w2_ref, b2_ref, out_ref):
    h = jnp.dot(table_ref[...], w1_ref[...], preferred_element_type=jnp.float32)
    h = h + b1_ref[...]
    h = h * jax.nn.sigmoid(h)
    out_ref[...] = (
        jnp.dot(h, w2_ref[...], preferred_element_type=jnp.float32) + b2_ref[...]
    )


def _mlp_table(table, W1, b1, W2, b2):
    return pl.pallas_call(
        _mlp_body,
        out_shape=jax.ShapeDtypeStruct((_NUM_TYPES, _EMBED), jnp.float32),
    )(table, W1, b1.reshape(1, _HIDDEN), W2, b2.reshape(1, _EMBED))


def _gather_body(tab_hbm, idx_hbm, out_hbm, tab_v, idx_v, *bufs):
    rows = bufs[:_NBUF]
    ssem = bufs[_NBUF:]

    wid = lax.axis_index("s") * _NC + lax.axis_index("c")
    base = wid * _BPW
    pltpu.sync_copy(tab_hbm, tab_v)
    pltpu.sync_copy(idx_hbm.at[pl.ds(base, _BPW)], idx_v)

    col_iota = lax.broadcasted_iota(jnp.int32, (_LANES,), 0)

    def dst(c):
        return out_hbm.at[pl.ds((base + c * _CHUNK) * _EMBED, _CHUNK * _EMBED)]

    def sstart(c):
        pltpu.async_copy(rows[c % _NBUF], dst(c), ssem[c % _NBUF])

    def swait(c):
        pltpu.make_async_copy(rows[c % _NBUF], dst(c), ssem[c % _NBUF]).wait()

    def compute_chunk(c, buf):
        def row_body(i, carry):
            abs_row = c * _CHUNK + i
            idx16 = idx_v[pl.ds((abs_row // _LANES) * _LANES, _LANES)]
            lane = jnp.full((_LANES,), abs_row % _LANES, dtype=jnp.int32)
            rowid = jnp.take_along_axis(idx16, lane, axis=0)
            ptr = rowid * _EMBED + col_iota
            for k in range(_NSEG):
                vals = plsc.load_gather(tab_v, [ptr])
                buf[pl.ds(i * _EMBED + k * _LANES, _LANES)] = vals
                if k + 1 < _NSEG:
                    ptr = ptr + _LANES
            return carry

        lax.fori_loop(0, _CHUNK, row_body, 0)

    for c in range(_NCHUNK):
        if c >= _NBUF:
            swait(c - _NBUF)
        compute_chunk(c, rows[c % _NBUF])
        sstart(c)
    for c in range(_NCHUNK - _NBUF, _NCHUNK):
        swait(c)


def _gather(out_table_flat, tok):
    mesh = plsc.VectorSubcoreMesh(
        core_axis_name="c", subcore_axis_name="s", num_cores=_NC
    )
    run = pl.kernel(
        _gather_body,
        out_type=jax.ShapeDtypeStruct((_B * _EMBED,), jnp.float32),
        mesh=mesh,
        compiler_params=pltpu.CompilerParams(needs_layout_passes=False),
        scratch_types=(
            [
                pltpu.VMEM((_NUM_TYPES * _EMBED,), jnp.float32),
                pltpu.VMEM((_BPW,), jnp.int32),
            ]
            + [pltpu.VMEM((_CHUNK * _EMBED,), jnp.float32) for _ in range(_NBUF)]
            + [pltpu.SemaphoreType.DMA for _ in range(_NBUF)]
        ),
    )
    return run(out_table_flat, tok)


def kernel(goal_tokens, table, W1, b1, W2, b2):
    tok = goal_tokens.astype(jnp.int32)
    out_table = _mlp_table(table, W1, b1, W2, b2)
    out_flat = _gather(out_table.reshape(-1), tok)
    return out_flat.reshape(_B, _EMBED)


# speedup vs baseline: 2.5996x; 2.5996x over previous
import jax
import jax.numpy as jnp
from jax import lax
from jax.experimental import pallas as pl
from jax.experimental.pallas import tpu as pltpu
from jax.experimental.pallas import tpu_sc as plsc

_NUM_TYPES = 16
_HIDDEN = 512
_EMBED = 768
_B = 16384

_NC = 2
_NS = 16
_NW = _NC * _NS
_BPW = _B // _NW
_NBUF = 4
_CHUNK = 32
_NCHUNK = _BPW // _CHUNK
_LANES = 16
_NREP = _NW


def _mlp_body(table_ref, w1_ref, b1_ref, w2_ref, b2_ref, out_ref):
    h = jnp.dot(table_ref[...], w1_ref[...], preferred_element_type=jnp.float32)
    h = h + b1_ref[...]
    h = h * jax.nn.sigmoid(h)
    o = jnp.dot(h, w2_ref[...], preferred_element_type=jnp.float32) + b2_ref[...]
    for r in range(_NREP):
        out_ref[pl.ds(r * _NUM_TYPES, _NUM_TYPES), :] = o


def _mlp_table(table, W1, b1, W2, b2):
    return pl.pallas_call(
        _mlp_body,
        out_shape=jax.ShapeDtypeStruct((_NREP * _NUM_TYPES, _EMBED), jnp.float32),
    )(table, W1, b1.reshape(1, _HIDDEN), W2, b2.reshape(1, _EMBED))


def _gather_body(tab_hbm, idx_hbm, out_hbm, idx_v, *bufs):
    rows = bufs[:_NBUF]
    gsem = bufs[_NBUF : 2 * _NBUF]
    ssem = bufs[2 * _NBUF :]

    wid = lax.axis_index("s") * _NC + lax.axis_index("c")
    base = wid * _BPW
    pltpu.sync_copy(idx_hbm.at[pl.ds(base, _BPW)], idx_v)

    off = jnp.full((_LANES,), wid * _NUM_TYPES, dtype=jnp.int32)
    for g in range(_BPW // _LANES):
        idx_v[pl.ds(g * _LANES, _LANES)] = idx_v[pl.ds(g * _LANES, _LANES)] + off

    def src(c):
        return tab_hbm.at[idx_v.at[pl.ds(c * _CHUNK, _CHUNK)]]

    def dst(c):
        return out_hbm.at[pl.ds(base + c * _CHUNK, _CHUNK)]

    def gstart(c):
        pltpu.async_copy(src(c), rows[c % _NBUF], gsem[c % _NBUF])

    def gwait(c):
        pltpu.make_async_copy(src(c), rows[c % _NBUF], gsem[c % _NBUF]).wait()

    def sstart(c):
        pltpu.async_copy(rows[c % _NBUF], dst(c), ssem[c % _NBUF])

    def swait(c):
        pltpu.make_async_copy(rows[c % _NBUF], dst(c), ssem[c % _NBUF]).wait()

    for c in range(_NBUF):
        gstart(c)
    for c in range(_NCHUNK):
        gwait(c)
        sstart(c)
        if c + _NBUF < _NCHUNK:
            swait(c)
            gstart(c + _NBUF)
    for c in range(_NCHUNK - _NBUF, _NCHUNK):
        swait(c)


def _gather(out_table, tok):
    mesh = plsc.VectorSubcoreMesh(
        core_axis_name="c", subcore_axis_name="s", num_cores=_NC
    )
    run = pl.kernel(
        _gather_body,
        out_type=jax.ShapeDtypeStruct((_B, _EMBED), jnp.float32),
        mesh=mesh,
        compiler_params=pltpu.CompilerParams(needs_layout_passes=False),
        scratch_types=(
            [pltpu.VMEM((_BPW,), jnp.int32)]
            + [pltpu.VMEM((_CHUNK, _EMBED), jnp.float32) for _ in range(_NBUF)]
            + [pltpu.SemaphoreType.DMA for _ in range(2 * _NBUF)]
        ),
    )
    return run(out_table, tok)


def kernel(goal_tokens, table, W1, b1, W2, b2):
    tok = goal_tokens.astype(jnp.int32)
    out_table = _mlp_table(table, W1, b1, W2, b2)
    return _gather(out_table, tok)


# speedup vs baseline: 2.7704x; 1.0657x over previous
import jax
import jax.numpy as jnp
from jax import lax
from jax.experimental import pallas as pl
from jax.experimental.pallas import tpu as pltpu
from jax.experimental.pallas import tpu_sc as plsc

_NUM_TYPES = 16
_HIDDEN = 512
_EMBED = 768
_B = 16384

_NC = 2
_NS = 16
_NW = _NC * _NS
_BPW = _B // _NW
_NBUF = 4
_CHUNK = 32
_NCHUNK = _BPW // _CHUNK
_LANES = 16
_REP_PER_TILE = 4
_NREP = _NW * _REP_PER_TILE


def _mlp_body(table_ref, w1_ref, b1_ref, w2_ref, b2_ref, out_ref):
    h = jnp.dot(table_ref[...], w1_ref[...], preferred_element_type=jnp.float32)
    h = h + b1_ref[...]
    h = h * jax.nn.sigmoid(h)
    o = jnp.dot(h, w2_ref[...], preferred_element_type=jnp.float32) + b2_ref[...]
    for r in range(_NREP):
        out_ref[pl.ds(r * _NUM_TYPES, _NUM_TYPES), :] = o


def _mlp_table(table, W1, b1, W2, b2):
    return pl.pallas_call(
        _mlp_body,
        out_shape=jax.ShapeDtypeStruct((_NREP * _NUM_TYPES, _EMBED), jnp.float32),
    )(table, W1, b1.reshape(1, _HIDDEN), W2, b2.reshape(1, _EMBED))


def _gather_body(tab_hbm, idx_hbm, out_hbm, idx_v, *bufs):
    rows = bufs[:_NBUF]
    gsem = bufs[_NBUF : 2 * _NBUF]
    ssem = bufs[2 * _NBUF :]

    wid = lax.axis_index("s") * _NC + lax.axis_index("c")
    base = wid * _BPW
    pltpu.sync_copy(idx_hbm.at[pl.ds(base, _BPW)], idx_v)

    groups_per_chunk = _CHUNK // _LANES
    for g in range(_BPW // _LANES):
        rep = wid * _REP_PER_TILE + (g // groups_per_chunk) % _REP_PER_TILE
        off = jnp.full((_LANES,), rep * _NUM_TYPES, dtype=jnp.int32)
        idx_v[pl.ds(g * _LANES, _LANES)] = idx_v[pl.ds(g * _LANES, _LANES)] + off

    def src(c):
        return tab_hbm.at[idx_v.at[pl.ds(c * _CHUNK, _CHUNK)]]

    def dst(c):
        return out_hbm.at[pl.ds(base + c * _CHUNK, _CHUNK)]

    def gstart(c):
        pltpu.async_copy(src(c), rows[c % _NBUF], gsem[c % _NBUF])

    def gwait(c):
        pltpu.make_async_copy(src(c), rows[c % _NBUF], gsem[c % _NBUF]).wait()

    def sstart(c):
        pltpu.async_copy(rows[c % _NBUF], dst(c), ssem[c % _NBUF])

    def swait(c):
        pltpu.make_async_copy(rows[c % _NBUF], dst(c), ssem[c % _NBUF]).wait()

    for c in range(_NBUF):
        gstart(c)
    for c in range(_NCHUNK):
        gwait(c)
        sstart(c)
        if c + _NBUF < _NCHUNK:
            swait(c)
            gstart(c + _NBUF)
    for c in range(_NCHUNK - _NBUF, _NCHUNK):
        swait(c)


def _gather(out_table, tok):
    mesh = plsc.VectorSubcoreMesh(
        core_axis_name="c", subcore_axis_name="s", num_cores=_NC
    )
    run = pl.kernel(
        _gather_body,
        out_type=jax.ShapeDtypeStruct((_B, _EMBED), jnp.float32),
        mesh=mesh,
        compiler_params=pltpu.CompilerParams(needs_layout_passes=False),
        scratch_types=(
            [pltpu.VMEM((_BPW,), jnp.int32)]
            + [pltpu.VMEM((_CHUNK, _EMBED), jnp.float32) for _ in range(_NBUF)]
            + [pltpu.SemaphoreType.DMA for _ in range(2 * _NBUF)]
        ),
    )
    return run(out_table, tok)


def kernel(goal_tokens, table, W1, b1, W2, b2):
    tok = goal_tokens.astype(jnp.int32)
    out_table = _mlp_table(table, W1, b1, W2, b2)
    return _gather(out_table, tok)
